# baseline (speedup 1.0000x reference)
import functools

import jax
import jax.numpy as jnp
from jax import lax
from jax.experimental import pallas as pl
from jax.experimental.pallas import tpu as pltpu
from jax.experimental.pallas import tpu_sc as plsc

NVERTS = 100000
BC = 14336
DTC = 2

BAND = 3072
REM = NVERTS - 32 * BAND
L_ = 512


def _tc_body(xt_ref, vs_ref, out_ref):
    L = xt_ref.shape[2]
    j = pl.program_id(1)
    out_ref[...] = jnp.zeros_like(out_ref)

    @pl.when(j == 0)
    def _scatter():
        col = jax.lax.broadcasted_iota(jnp.int32, (L, L), 1)
        q = (vs_ref[0][:, None] == col).astype(jnp.float32)
        out_ref[0, :, :L] = jnp.dot(
            xt_ref[0], q, preferred_element_type=jnp.float32,
            precision=jax.lax.Precision.HIGHEST)


def _make_sc(R, NG):
    mesh = plsc.VectorSubcoreMesh(core_axis_name="c", subcore_axis_name="s")

    @functools.partial(
        pl.kernel,
        mesh=mesh,
        out_type=jax.ShapeDtypeStruct((R, NVERTS), jnp.float32),
        scratch_types=[
            pltpu.VMEM((8, BAND), jnp.float32),
            pltpu.VMEM((8, REM), jnp.float32),
            pltpu.VMEM((8, BAND - L_), jnp.float32),
            pltpu.VMEM((2, 8, L_), jnp.float32),
            pltpu.SemaphoreType.DMA,
            pltpu.SemaphoreType.DMA,
            pltpu.SemaphoreType.DMA,
        ],
    )
    def sc_k(xt_hbm, vs_hbm, out_hbm, zband, zrem, zht, xrows,
             dsem, hsem, xsem):
        wid = lax.axis_index("s") * 2 + lax.axis_index("c")
        col0 = pl.multiple_of(wid * BAND, 128)

        zero16 = jnp.zeros((16,), jnp.float32)

        for rr in range(8):
            zb = zband.at[rr]
            zr = zrem.at[rr]
            zh = zht.at[rr]

            def zf_band(i, c):
                zb[pl.ds(i * 16, 16)] = zero16
                return c

            lax.fori_loop(0, BAND // 16, zf_band, 0, unroll=8)

            def zf_rem(i, c):
                zr[pl.ds(i * 16, 16)] = zero16
                return c

            lax.fori_loop(0, REM // 16, zf_rem, 0, unroll=8)

            def zf_ht(i, c):
                zh[pl.ds(i * 16, 16)] = zero16
                return c

            lax.fori_loop(0, (BAND - L_) // 16, zf_ht, 0, unroll=8)

        @pl.when(wid != 0)
        def _plain_bands():
            def band_g(g, c):
                r0 = pl.multiple_of(g * 8, 8)
                pltpu.async_copy(
                    zband,
                    out_hbm.at[pl.ds(r0, 8), pl.ds(col0, BAND)], dsem)
                return c

            lax.fori_loop(0, NG, band_g, 0)

        @pl.when(wid == 0)
        def _scatter_bands():
            pltpu.async_copy(xt_hbm.at[pl.ds(0, 8)], xrows.at[0], xsem)

            def group(g, c):
                r0 = pl.multiple_of(g * 8, 8)
                pg = g % 2
                pltpu.make_async_copy(
                    xt_hbm.at[pl.ds(0, 8)], xrows.at[0], xsem).wait()
                pltpu.async_copy(
                    xrows.at[pg],
                    out_hbm.at[pl.ds(r0, 8), pl.ds(0, L_)], hsem)
                pltpu.async_copy(
                    zht,
                    out_hbm.at[pl.ds(r0, 8), pl.ds(L_, BAND - L_)], dsem)

                @pl.when(g >= 1)
                def _drain_prev_head():
                    pltpu.make_async_copy(
                        xrows.at[0],
                        out_hbm.at[pl.ds(0, 8), pl.ds(0, L_)], hsem).wait()

                @pl.when(g + 1 < NG)
                def _prefetch():
                    r1 = pl.multiple_of((g + 1) * 8, 8)
                    pltpu.async_copy(
                        xt_hbm.at[pl.ds(r1, 8)], xrows.at[1 - pg], xsem)
                return c

            lax.fori_loop(0, NG, group, 0)
            pltpu.make_async_copy(
                xrows.at[0],
                out_hbm.at[pl.ds(0, 8), pl.ds(0, L_)], hsem).wait()

        @pl.when(jnp.logical_and(wid >= 8, wid < 8 + NG))
        def _rem_band():
            r0 = pl.multiple_of((wid - 8) * 8, 8)
            pltpu.async_copy(
                zrem,
                out_hbm.at[pl.ds(r0, 8), pl.ds(32 * BAND, REM)], dsem)

        @pl.when(wid != 0)
        def _drain_bands():
            def dr(g, c):
                pltpu.make_async_copy(
                    zband,
                    out_hbm.at[pl.ds(0, 8), pl.ds(0, BAND)], dsem).wait()
                return c

            lax.fori_loop(0, NG, dr, 0)

        @pl.when(wid == 0)
        def _drain_tails():
            def dr(g, c):
                pltpu.make_async_copy(
                    zht,
                    out_hbm.at[pl.ds(0, 8), pl.ds(L_, BAND - L_)],
                    dsem).wait()
                return c

            lax.fori_loop(0, NG, dr, 0)

        @pl.when(jnp.logical_and(wid >= 8, wid < 8 + NG))
        def _drain_rem():
            pltpu.make_async_copy(
                zrem,
                out_hbm.at[pl.ds(0, 8), pl.ds(32 * BAND, REM)], dsem).wait()

    return sc_k


def kernel(x, vs):
    B, L, D = x.shape
    xt = jnp.transpose(x, (2, 0, 1))
    vs2 = vs.reshape(1, L)

    tc_out = pl.pallas_call(
        _tc_body,
        grid=(DTC, pl.cdiv(NVERTS, BC)),
        in_specs=[
            pl.BlockSpec((1, B, L), lambda d, j: (d, 0, 0)),
            pl.BlockSpec((1, L), lambda d, j: (0, 0)),
        ],
        out_specs=pl.BlockSpec((1, B, BC), lambda d, j: (d, 0, j)),
        out_shape=jax.ShapeDtypeStruct((DTC, B, NVERTS), jnp.float32),
        compiler_params=pltpu.CompilerParams(
            dimension_semantics=("parallel", "parallel")),
    )(xt[:DTC], vs2)

    RSC = (D - DTC) * B
    sc_k = _make_sc(RSC, RSC // 8)
    sc_out = sc_k(xt[DTC:].reshape(RSC, L), vs)

    out = jnp.concatenate(
        [tc_out.reshape(DTC * B, NVERTS), sc_out], axis=0)
    return jnp.transpose(out.reshape(D, B, NVERTS), (1, 2, 0))
